# Initial kernel scaffold; baseline (speedup 1.0000x reference)
import jax
import jax.numpy as jnp
from jax.experimental import pallas as pl


def kernel(vertices, dir0, w1, b1, d1, w2, b2, d2, w3, b3, d3, w4, b4, d4, Wl, bl):
    raise NotImplementedError("write your pallas kernel here")


# speedup vs baseline: 30.2195x; 30.2195x over previous
import functools
import math

import numpy as _np

import jax
import jax.numpy as jnp
from jax import lax
from jax.experimental import pallas as pl
from jax.experimental.pallas import tpu as pltpu
from jax.experimental.pallas import tpu_sc as plsc

_SUP = 3
_NBR = 16
_PNBR = 4
_F32 = jnp.float32
_HI = lax.Precision.HIGHEST
_Z = _np.int32(0)


def _gather_rows(table, idx):
    T, D = table.shape
    (Btot,) = idx.shape
    info = plsc.get_sparse_core_info()
    NC, NS = info.num_cores, info.num_subcores
    NW = NC * NS
    assert Btot % (8 * NW) == 0 and D % 128 == 0
    rpw = Btot // NW
    cap = max(8, 400_000 // (4 * D))
    chunk = 8
    while chunk * 2 <= min(rpw, cap, 1024):
        chunk *= 2
    nchunks = rpw // chunk
    mesh = plsc.VectorSubcoreMesh(core_axis_name="c", subcore_axis_name="s")

    @functools.partial(
        pl.kernel,
        mesh=mesh,
        out_type=jax.ShapeDtypeStruct((Btot, D), _F32),
        scratch_types=[
            pltpu.VMEM((chunk,), jnp.int32),
            pltpu.VMEM((chunk, D), _F32),
            pltpu.SemaphoreType.DMA,
        ],
    )
    def gk(table_hbm, idx_hbm, out_hbm, idx_v, rows_v, sem):
        i32 = jnp.int32
        wid = lax.axis_index("s") * i32(NC) + lax.axis_index("c")
        base0 = wid * i32(rpw)

        def body(c, carry):
            base = base0 + c * i32(chunk)
            pltpu.sync_copy(idx_hbm.at[pl.ds(base, chunk)], idx_v)
            pltpu.async_copy(table_hbm.at[idx_v], rows_v, sem).wait()
            pltpu.sync_copy(rows_v, out_hbm.at[pl.ds(base, chunk)])
            return carry

        lax.fori_loop(jnp.int32(0), jnp.int32(nchunks), body, jnp.int32(0))

    return gk(table, idx)


def _knn_global(vq, vtT, k, rq):
    B, Nq, _ = vq.shape
    Nt = vtT.shape[2]

    def body(vq_ref, vt_ref, o_ref):
        b = pl.program_id(0)
        q = vq_ref[0]
        t = vt_ref[0]
        inner = jnp.dot(q.astype(jnp.bfloat16), t.astype(jnp.bfloat16),
                        preferred_element_type=_F32)
        qq = q[:, 0:1] * q[:, 0:1]
        qt = t[0:1, :] * t[0:1, :]
        for d in (1, 2):
            qq = qq + q[:, d:d + 1] * q[:, d:d + 1]
            qt = qt + t[d:d + 1, :] * t[d:d + 1, :]
        dist = -2.0 * inner + qt + qq
        iota = lax.broadcasted_iota(jnp.int32, (rq, Nt), 1)
        big = jnp.int32(Nt)
        cols = []
        for _ in range(k):
            m = jnp.min(dist, axis=1, keepdims=True)
            am = jnp.min(jnp.where(dist <= m, iota, big), axis=1, keepdims=True)
            cols.append(am + b * Nt)
            dist = jnp.where(iota == am, _F32(jnp.inf), dist)
        o_ref[0] = jnp.concatenate(cols, axis=1)

    return pl.pallas_call(
        body,
        grid=(B, Nq // rq),
        in_specs=[
            pl.BlockSpec((1, rq, 3), lambda b, i: (b, i, _Z)),
            pl.BlockSpec((1, 3, Nt), lambda b, i: (b, _Z, _Z)),
        ],
        out_specs=pl.BlockSpec((1, rq, k), lambda b, i: (b, i, _Z)),
        out_shape=jax.ShapeDtypeStruct((B, Nq, k), jnp.int32),
    )(vq, vtT)


def _mm(x, w, b2d, rm):
    Rt, K = x.shape
    D = w.shape[1]

    def body(x_ref, w_ref, b_ref, o_ref):
        o_ref[...] = (
            jnp.dot(x_ref[...], w_ref[...], precision=_HI,
                    preferred_element_type=_F32)
            + b_ref[...]
        )

    return pl.pallas_call(
        body,
        grid=(Rt // rm,),
        in_specs=[
            pl.BlockSpec((rm, K), lambda i: (i, _Z)),
            pl.BlockSpec((K, D), lambda i: (_Z, _Z)),
            pl.BlockSpec((1, D), lambda i: (_Z, _Z)),
        ],
        out_specs=pl.BlockSpec((rm, D), lambda i: (i, _Z)),
        out_shape=jax.ShapeDtypeStruct((Rt, D), _F32),
    )(x, w, b2d)


def _dirs_norm(dirs):
    n2 = dirs[0:1, :] * dirs[0:1, :]
    for d in (1, 2):
        n2 = n2 + dirs[d:d + 1, :] * dirs[d:d + 1, :]
    return dirs / jnp.maximum(jnp.sqrt(n2), _F32(1e-12))


def _theta_j(nb3, vq3, sdn, j, R, D):
    d = nb3[:, j, 0:3] - vq3
    n2 = jnp.sum(d * d, axis=1, keepdims=True)
    dn = d / jnp.maximum(jnp.sqrt(n2), _F32(1e-12))
    th = jnp.dot(dn, sdn, precision=_HI, preferred_element_type=_F32)
    return jnp.maximum(th, _F32(0.0))


def _conv_surface(nbv, vq, dirs, R, kn):
    Rt = vq.shape[0]
    D = dirs.shape[1]

    def body(nbv_ref, vq_ref, dir_ref, o_ref):
        sdn = _dirs_norm(dir_ref[...])
        vq3 = vq_ref[...][:, 0:3]
        nb3 = nbv_ref[...].reshape(R, _NBR, 128)
        m = _theta_j(nb3, vq3, sdn, 0, R, D)
        for j in range(1, _NBR):
            m = jnp.maximum(m, _theta_j(nb3, vq3, sdn, j, R, D))
        acc = m[:, 0:kn]
        for s in range(1, _SUP):
            acc = acc + m[:, s * kn:(s + 1) * kn]
        o_ref[...] = jnp.maximum(acc, _F32(0.0))

    return pl.pallas_call(
        body,
        grid=(Rt // R,),
        in_specs=[
            pl.BlockSpec((R * _NBR, 128), lambda i: (i, _Z)),
            pl.BlockSpec((R, 128), lambda i: (i, _Z)),
            pl.BlockSpec((3, D), lambda i: (_Z, _Z)),
        ],
        out_specs=pl.BlockSpec((R, kn), lambda i: (i, _Z)),
        out_shape=jax.ShapeDtypeStruct((Rt, kn), _F32),
    )(nbv, vq, dirs)


def _conv_layer(fc, fs, nbv, vq, dirs, R, out, do_relu):
    Rt = vq.shape[0]
    D = dirs.shape[1]
    Dfull = fs.shape[1]

    def body(fc_ref, fs_ref, nbv_ref, vq_ref, dir_ref, o_ref):
        sdn = _dirs_norm(dir_ref[...])
        vq3 = vq_ref[...][:, 0:3]
        nb3 = nbv_ref[...].reshape(R, _NBR, 128)
        fs3 = fs_ref[...].reshape(R, _NBR, Dfull)
        m = _theta_j(nb3, vq3, sdn, 0, R, D) * fs3[:, 0, 0:D]
        for j in range(1, _NBR):
            m = jnp.maximum(
                m, _theta_j(nb3, vq3, sdn, j, R, D) * fs3[:, j, 0:D])
        acc = fc_ref[...] + m[:, 0:out]
        for s in range(1, _SUP):
            acc = acc + m[:, s * out:(s + 1) * out]
        if do_relu:
            acc = jnp.maximum(acc, _F32(0.0))
        o_ref[...] = acc

    return pl.pallas_call(
        body,
        grid=(Rt // R,),
        in_specs=[
            pl.BlockSpec((R, out), lambda i: (i, _Z)),
            pl.BlockSpec((R * _NBR, Dfull), lambda i: (i, _Z)),
            pl.BlockSpec((R * _NBR, 128), lambda i: (i, _Z)),
            pl.BlockSpec((R, 128), lambda i: (i, _Z)),
            pl.BlockSpec((3, D), lambda i: (_Z, _Z)),
        ],
        out_specs=pl.BlockSpec((R, out), lambda i: (i, _Z)),
        out_shape=jax.ShapeDtypeStruct((Rt, out), _F32),
    )(fc, fs, nbv, vq, dirs)


def _maxpool4(rows, R):
    Rt4, D = rows.shape
    Rt = Rt4 // _PNBR

    def body(x_ref, o_ref):
        x3 = x_ref[...].reshape(R, _PNBR, D)
        m = x3[:, 0, :]
        for j in range(1, _PNBR):
            m = jnp.maximum(m, x3[:, j, :])
        o_ref[...] = m

    return pl.pallas_call(
        body,
        grid=(Rt // R,),
        in_specs=[pl.BlockSpec((R * _PNBR, D), lambda i: (i, _Z))],
        out_specs=pl.BlockSpec((R, D), lambda i: (i, _Z)),
        out_shape=jax.ShapeDtypeStruct((Rt, D), _F32),
    )(rows)


def _final(fm4, WlT, bl2d, B, N):
    D = fm4.shape[1]
    O = WlT.shape[1]

    def body(x_ref, w_ref, b_ref, o_ref):
        x3 = x_ref[...].reshape(B, N, D)
        fg = jnp.max(x3, axis=1)
        o_ref[...] = (
            jnp.dot(fg, w_ref[...], precision=_HI, preferred_element_type=_F32)
            + b_ref[...]
        )

    return pl.pallas_call(
        body,
        in_specs=[
            pl.BlockSpec((B * N, D), lambda: (_Z, _Z)),
            pl.BlockSpec((D, O), lambda: (_Z, _Z)),
            pl.BlockSpec((1, O), lambda: (_Z, _Z)),
        ],
        out_specs=pl.BlockSpec((B, O), lambda: (_Z, _Z)),
        out_shape=jax.ShapeDtypeStruct((B, O), _F32),
    )(fm4, WlT, bl2d)


def _padw(flat, w):
    R, d = flat.shape
    return jnp.concatenate([flat, jnp.zeros((R, w - d), _F32)], axis=1)


def kernel(vertices, dir0, w1, b1, d1, w2, b2, d2, w3, b3, d3, w4, b4, d4,
           Wl, bl):
    B, N0, _ = vertices.shape
    N1, N2 = N0 // 4, N0 // 16
    f32 = lambda x: x.astype(_F32)
    vertices = f32(vertices)

    sel1 = jax.random.permutation(jax.random.key(1), N0)[:N1].astype(jnp.int32)
    sel2 = jax.random.permutation(jax.random.key(2), N1)[:N2].astype(jnp.int32)
    boff = lambda n: (jnp.arange(B, dtype=jnp.int32) * n)[:, None]
    sel1_g = (sel1[None, :] + boff(N0)).reshape(-1)
    sel2_g = (sel2[None, :] + boff(N1)).reshape(-1)

    def reorder(w, b, out):
        return (f32(jnp.concatenate([w[:, out:], w[:, :out]], axis=1)),
                f32(jnp.concatenate([b[out:], b[:out]])).reshape(1, -1))

    w1r, b1r = reorder(w1, b1, 64)
    w2r, b2r = reorder(w2, b2, 128)
    w3r, b3r = reorder(w3, b3, 256)
    w4r, b4r = reorder(w4, b4, 1024)

    vpad0 = _padw(vertices.reshape(B * N0, 3), 128)
    vtT0 = jnp.transpose(vertices, (0, 2, 1))

    nbr0 = _knn_global(vertices, vtT0, _NBR + 1, 256)[:, :, 1:]
    idx0 = nbr0.reshape(-1)
    nbv0 = _gather_rows(vpad0, idx0)

    fm0 = _conv_surface(nbv0, vpad0, f32(dir0), 512, 32)
    fo1 = _mm(fm0, w1r, b1r, 1024)
    fc1 = fo1[:, 192:]
    fs1 = _gather_rows(fo1, idx0)
    fm1 = _conv_layer(fc1, fs1, nbv0, vpad0, f32(d1), 256, 64, True)

    v1pad = _gather_rows(vpad0, sel1_g)
    v1 = v1pad[:, :3].reshape(B, N1, 3)
    nbrp1 = _knn_global(v1, vtT0, _PNBR + 1, 256)[:, :, 1:]
    prow1 = _gather_rows(_padw(fm1, 128), nbrp1.reshape(-1))
    fm1p = _maxpool4(prow1, 512)[:, :64]

    vtT1 = jnp.transpose(v1, (0, 2, 1))
    nbr1 = _knn_global(v1, vtT1, _NBR + 1, 256)[:, :, 1:]
    idx1 = nbr1.reshape(-1)
    nbv1 = _gather_rows(v1pad, idx1)

    fo2 = _mm(fm1p, w2r, b2r, 1024)
    fc2 = fo2[:, 384:]
    fs2 = _gather_rows(fo2, idx1)
    fm2 = _conv_layer(fc2, fs2, nbv1, v1pad, f32(d2), 256, 128, True)

    fo3 = _mm(fm2, w3r, b3r, 1024)
    fc3 = fo3[:, 768:]
    fs3 = _gather_rows(fo3, idx1)
    fm3 = _conv_layer(fc3, fs3, nbv1, v1pad, f32(d3), 128, 256, True)

    v2pad = _gather_rows(v1pad, sel2_g)
    v2 = v2pad[:, :3].reshape(B, N2, 3)
    nbrp2 = _knn_global(v2, vtT1, _PNBR + 1, 256)[:, :, 1:]
    prow2 = _gather_rows(fm3, nbrp2.reshape(-1))
    fm3p = _maxpool4(prow2, 512)

    vtT2 = jnp.transpose(v2, (0, 2, 1))
    nbr2 = _knn_global(v2, vtT2, _NBR + 1, 256)[:, :, 1:]
    idx2 = nbr2.reshape(-1)
    nbv2 = _gather_rows(v2pad, idx2)

    fo4 = _mm(fm3p, w4r, b4r, 512)
    fc4 = fo4[:, 3072:]
    fs4 = _gather_rows(fo4, idx2)
    fm4 = _conv_layer(fc4, fs4, nbv2, v2pad, f32(d4), 32, 1024, False)

    return _final(fm4, f32(Wl).T, f32(bl).reshape(1, -1), B, N2)


# speedup vs baseline: 43.0645x; 1.4251x over previous
import functools
import math

import numpy as _np

import jax
import jax.numpy as jnp
from jax import lax
from jax.experimental import pallas as pl
from jax.experimental.pallas import tpu as pltpu
from jax.experimental.pallas import tpu_sc as plsc

_SUP = 3
_NBR = 16
_PNBR = 4
_F32 = jnp.float32
_HI = lax.Precision.HIGHEST
_Z = _np.int32(0)


def _gather_rows(table, idx):
    T, D = table.shape
    (Btot,) = idx.shape
    info = plsc.get_sparse_core_info()
    NC, NS = info.num_cores, info.num_subcores
    NW = NC * NS
    assert Btot % (8 * NW) == 0 and D % 128 == 0
    rpw = Btot // NW
    cap = max(8, 180_000 // (4 * D))
    chunk = 8
    while chunk * 2 <= min(rpw, cap, 1024):
        chunk *= 2
    nchunks = rpw // chunk
    mesh = plsc.VectorSubcoreMesh(core_axis_name="c", subcore_axis_name="s")

    @functools.partial(
        pl.kernel,
        mesh=mesh,
        out_type=jax.ShapeDtypeStruct((Btot, D), _F32),
        scratch_types=[
            pltpu.VMEM((chunk,), jnp.int32),
            pltpu.VMEM((chunk,), jnp.int32),
            pltpu.VMEM((chunk, D), _F32),
            pltpu.VMEM((chunk, D), _F32),
            pltpu.SemaphoreType.DMA,
            pltpu.SemaphoreType.DMA,
            pltpu.SemaphoreType.DMA,
            pltpu.SemaphoreType.DMA,
        ],
    )
    def gk(table_hbm, idx_hbm, out_hbm, idx_a, idx_b, rows_a, rows_b,
           sga, sgb, swa, swb):
        i32 = jnp.int32
        wid = lax.axis_index("s") * i32(NC) + lax.axis_index("c")
        base0 = wid * i32(rpw)
        idx_v = (idx_a, idx_b)
        rows_v = (rows_a, rows_b)
        sg = (sga, sgb)
        sw = (swa, swb)

        def start_gather(c):
            b = c % 2
            base = base0 + i32(c * chunk)
            pltpu.sync_copy(idx_hbm.at[pl.ds(base, chunk)], idx_v[b])
            return pltpu.async_copy(table_hbm.at[idx_v[b]], rows_v[b], sg[b])

        gh = {0: start_gather(0)}
        if nchunks > 1:
            gh[1] = start_gather(1)
        wh = {}
        for c in range(nchunks):
            b = c % 2
            gh[c].wait()
            base = base0 + i32(c * chunk)
            wh[c] = pltpu.async_copy(rows_v[b], out_hbm.at[pl.ds(base, chunk)],
                                     sw[b])
            if c + 2 < nchunks:
                wh[c].wait()
                gh[c + 2] = start_gather(c + 2)
        for c in (nchunks - 2, nchunks - 1):
            if c >= 0 and c in wh and c + 2 >= nchunks:
                wh[c].wait()

    return gk(table, idx)


def _knn_global(vq, vtT, k, rq):
    B, Nq, _ = vq.shape
    Nt = vtT.shape[2]

    def body(vq_ref, vt_ref, o_ref):
        b = pl.program_id(0)
        q = vq_ref[0]
        t = vt_ref[0]
        inner = jnp.dot(q.astype(jnp.bfloat16), t.astype(jnp.bfloat16),
                        preferred_element_type=_F32)
        qq = q[:, 0:1] * q[:, 0:1]
        qt = t[0:1, :] * t[0:1, :]
        for d in (1, 2):
            qq = qq + q[:, d:d + 1] * q[:, d:d + 1]
            qt = qt + t[d:d + 1, :] * t[d:d + 1, :]
        dist = -2.0 * inner + qt + qq
        iota = lax.broadcasted_iota(jnp.int32, (rq, Nt), 1)
        big = jnp.int32(Nt)
        cols = []
        for _ in range(k):
            m = jnp.min(dist, axis=1, keepdims=True)
            am = jnp.min(jnp.where(dist <= m, iota, big), axis=1, keepdims=True)
            cols.append(am + b * Nt)
            dist = jnp.where(iota == am, _F32(jnp.inf), dist)
        o_ref[0] = jnp.concatenate(cols, axis=1)

    return pl.pallas_call(
        body,
        grid=(B, Nq // rq),
        in_specs=[
            pl.BlockSpec((1, rq, 3), lambda b, i: (b, i, _Z)),
            pl.BlockSpec((1, 3, Nt), lambda b, i: (b, _Z, _Z)),
        ],
        out_specs=pl.BlockSpec((1, rq, k), lambda b, i: (b, i, _Z)),
        out_shape=jax.ShapeDtypeStruct((B, Nq, k), jnp.int32),
    )(vq, vtT)


def _mm(x, w, b2d, rm):
    Rt, K = x.shape
    D = w.shape[1]

    def body(x_ref, w_ref, b_ref, o_ref):
        o_ref[...] = (
            jnp.dot(x_ref[...], w_ref[...], precision=_HI,
                    preferred_element_type=_F32)
            + b_ref[...]
        )

    return pl.pallas_call(
        body,
        grid=(Rt // rm,),
        in_specs=[
            pl.BlockSpec((rm, K), lambda i: (i, _Z)),
            pl.BlockSpec((K, D), lambda i: (_Z, _Z)),
            pl.BlockSpec((1, D), lambda i: (_Z, _Z)),
        ],
        out_specs=pl.BlockSpec((rm, D), lambda i: (i, _Z)),
        out_shape=jax.ShapeDtypeStruct((Rt, D), _F32),
    )(x, w, b2d)


def _dirs_norm(dirs):
    n2 = dirs[0:1, :] * dirs[0:1, :]
    for d in (1, 2):
        n2 = n2 + dirs[d:d + 1, :] * dirs[d:d + 1, :]
    return dirs / jnp.maximum(jnp.sqrt(n2), _F32(1e-12))


def _theta_j(nb_j, vq3, sdn):
    d = nb_j[:, 0:3] - vq3
    n2 = jnp.sum(d * d, axis=1, keepdims=True)
    dn = d / jnp.maximum(jnp.sqrt(n2), _F32(1e-12))
    th = (dn[:, 0:1] * sdn[0:1, :] + dn[:, 1:2] * sdn[1:2, :]
          + dn[:, 2:3] * sdn[2:3, :])
    return jnp.maximum(th, _F32(0.0))


def _conv_surface(nbv, vq, dirs, R, kn):
    Rt = vq.shape[0]
    D = dirs.shape[1]

    def body(nbv_ref, vq_ref, dir_ref, o_ref):
        sdn = _dirs_norm(dir_ref[...])
        vq3 = vq_ref[...][:, 0:3]
        m = _theta_j(nbv_ref[0], vq3, sdn)
        for j in range(1, _NBR):
            m = jnp.maximum(m, _theta_j(nbv_ref[j], vq3, sdn))
        acc = m[:, 0:kn]
        for s in range(1, _SUP):
            acc = acc + m[:, s * kn:(s + 1) * kn]
        o_ref[...] = jnp.maximum(acc, _F32(0.0))

    return pl.pallas_call(
        body,
        grid=(Rt // R,),
        in_specs=[
            pl.BlockSpec((_NBR, R, 128), lambda i: (_Z, i, _Z)),
            pl.BlockSpec((R, 128), lambda i: (i, _Z)),
            pl.BlockSpec((3, D), lambda i: (_Z, _Z)),
        ],
        out_specs=pl.BlockSpec((R, kn), lambda i: (i, _Z)),
        out_shape=jax.ShapeDtypeStruct((Rt, kn), _F32),
    )(nbv, vq, dirs)


def _conv_layer(fc, fs, nbv, vq, dirs, R, out, do_relu):
    Rt = vq.shape[0]
    D = dirs.shape[1]
    Dfull = fs.shape[2]

    def body(fc_ref, fs_ref, nbv_ref, vq_ref, dir_ref, o_ref):
        sdn = _dirs_norm(dir_ref[...])
        vq3 = vq_ref[...][:, 0:3]
        m = _theta_j(nbv_ref[0], vq3, sdn) * fs_ref[0][:, 0:D]
        for j in range(1, _NBR):
            m = jnp.maximum(
                m, _theta_j(nbv_ref[j], vq3, sdn) * fs_ref[j][:, 0:D])
        acc = fc_ref[...] + m[:, 0:out]
        for s in range(1, _SUP):
            acc = acc + m[:, s * out:(s + 1) * out]
        if do_relu:
            acc = jnp.maximum(acc, _F32(0.0))
        o_ref[...] = acc

    return pl.pallas_call(
        body,
        grid=(Rt // R,),
        in_specs=[
            pl.BlockSpec((R, out), lambda i: (i, _Z)),
            pl.BlockSpec((_NBR, R, Dfull), lambda i: (_Z, i, _Z)),
            pl.BlockSpec((_NBR, R, 128), lambda i: (_Z, i, _Z)),
            pl.BlockSpec((R, 128), lambda i: (i, _Z)),
            pl.BlockSpec((3, D), lambda i: (_Z, _Z)),
        ],
        out_specs=pl.BlockSpec((R, out), lambda i: (i, _Z)),
        out_shape=jax.ShapeDtypeStruct((Rt, out), _F32),
    )(fc, fs, nbv, vq, dirs)


def _maxpool4(rows, R):
    _, Rt, D = rows.shape

    def body(x_ref, o_ref):
        m = x_ref[0]
        for j in range(1, _PNBR):
            m = jnp.maximum(m, x_ref[j])
        o_ref[...] = m

    return pl.pallas_call(
        body,
        grid=(Rt // R,),
        in_specs=[pl.BlockSpec((_PNBR, R, D), lambda i: (_Z, i, _Z))],
        out_specs=pl.BlockSpec((R, D), lambda i: (i, _Z)),
        out_shape=jax.ShapeDtypeStruct((Rt, D), _F32),
    )(rows)


def _final(fm4, WlT, bl2d, B, N):
    D = fm4.shape[1]
    O = WlT.shape[1]

    def body(x_ref, w_ref, b_ref, o_ref):
        x3 = x_ref[...].reshape(B, N, D)
        fg = jnp.max(x3, axis=1)
        o_ref[...] = (
            jnp.dot(fg, w_ref[...], precision=_HI, preferred_element_type=_F32)
            + b_ref[...]
        )

    return pl.pallas_call(
        body,
        in_specs=[
            pl.BlockSpec((B * N, D), lambda: (_Z, _Z)),
            pl.BlockSpec((D, O), lambda: (_Z, _Z)),
            pl.BlockSpec((1, O), lambda: (_Z, _Z)),
        ],
        out_specs=pl.BlockSpec((B, O), lambda: (_Z, _Z)),
        out_shape=jax.ShapeDtypeStruct((B, O), _F32),
    )(fm4, WlT, bl2d)


def _padw(flat, w):
    R, d = flat.shape
    return jnp.concatenate([flat, jnp.zeros((R, w - d), _F32)], axis=1)


def kernel(vertices, dir0, w1, b1, d1, w2, b2, d2, w3, b3, d3, w4, b4, d4,
           Wl, bl):
    B, N0, _ = vertices.shape
    N1, N2 = N0 // 4, N0 // 16
    f32 = lambda x: x.astype(_F32)
    vertices = f32(vertices)

    sel1 = jax.random.permutation(jax.random.key(1), N0)[:N1].astype(jnp.int32)
    sel2 = jax.random.permutation(jax.random.key(2), N1)[:N2].astype(jnp.int32)
    boff = lambda n: (jnp.arange(B, dtype=jnp.int32) * n)[:, None]
    sel1_g = (sel1[None, :] + boff(N0)).reshape(-1)
    sel2_g = (sel2[None, :] + boff(N1)).reshape(-1)

    w1r = f32(jnp.concatenate([w1[:, 64:], w1[:, :64]], axis=1))
    b1r = f32(jnp.concatenate([b1[64:], b1[:64]])).reshape(1, -1)

    def split(w, b, out):
        return (f32(w[:, :out]), f32(b[:out]).reshape(1, -1),
                f32(w[:, out:]), f32(b[out:]).reshape(1, -1))

    w2c, b2c, w2t, b2t = split(w2, b2, 128)
    w3c, b3c, w3t, b3t = split(w3, b3, 256)
    w4c, b4c, w4t, b4t = split(w4, b4, 1024)

    vpad0 = _padw(vertices.reshape(B * N0, 3), 128)
    vtT0 = jnp.transpose(vertices, (0, 2, 1))

    jmaj = lambda nbr: jnp.transpose(nbr, (2, 0, 1)).reshape(-1)

    nbr0 = _knn_global(vertices, vtT0, _NBR + 1, 256)[:, :, 1:]
    idx0 = jmaj(nbr0)
    nbv0 = _gather_rows(vpad0, idx0).reshape(_NBR, B * N0, 128)

    fm0 = _conv_surface(nbv0, vpad0, f32(dir0), 512, 32)
    fo1 = _mm(fm0, w1r, b1r, 1024)
    fc1 = fo1[:, 192:]
    fs1 = _gather_rows(fo1, idx0).reshape(_NBR, B * N0, 256)
    fm1 = _conv_layer(fc1, fs1, nbv0, vpad0, f32(d1), 256, 64, True)

    v1pad = _gather_rows(vpad0, sel1_g)
    v1 = v1pad[:, :3].reshape(B, N1, 3)
    nbrp1 = _knn_global(v1, vtT0, _PNBR + 1, 256)[:, :, 1:]
    prow1 = _gather_rows(_padw(fm1, 128), jmaj(nbrp1))
    fm1p = _maxpool4(prow1.reshape(_PNBR, B * N1, 128), 512)[:, :64]

    vtT1 = jnp.transpose(v1, (0, 2, 1))
    nbr1 = _knn_global(v1, vtT1, _NBR + 1, 256)[:, :, 1:]
    idx1 = jmaj(nbr1)
    nbv1 = _gather_rows(v1pad, idx1).reshape(_NBR, B * N1, 128)

    fc2 = _mm(fm1p, w2c, b2c, 1024)
    ft2 = _mm(fm1p, w2t, b2t, 1024)
    fs2 = _gather_rows(ft2, idx1).reshape(_NBR, B * N1, 384)
    fm2 = _conv_layer(fc2, fs2, nbv1, v1pad, f32(d2), 256, 128, True)

    fc3 = _mm(fm2, w3c, b3c, 1024)
    ft3 = _mm(fm2, w3t, b3t, 1024)
    fs3 = _gather_rows(ft3, idx1).reshape(_NBR, B * N1, 768)
    fm3 = _conv_layer(fc3, fs3, nbv1, v1pad, f32(d3), 128, 256, True)

    v2pad = _gather_rows(v1pad, sel2_g)
    v2 = v2pad[:, :3].reshape(B, N2, 3)
    nbrp2 = _knn_global(v2, vtT1, _PNBR + 1, 256)[:, :, 1:]
    prow2 = _gather_rows(fm3, jmaj(nbrp2))
    fm3p = _maxpool4(prow2.reshape(_PNBR, B * N2, 256), 512)

    vtT2 = jnp.transpose(v2, (0, 2, 1))
    nbr2 = _knn_global(v2, vtT2, _NBR + 1, 256)[:, :, 1:]
    idx2 = jmaj(nbr2)
    nbv2 = _gather_rows(v2pad, idx2).reshape(_NBR, B * N2, 128)

    fc4 = _mm(fm3p, w4c, b4c, 512)
    ft4 = _mm(fm3p, w4t, b4t, 512)
    fs4 = _gather_rows(ft4, idx2).reshape(_NBR, B * N2, 3072)
    fm4 = _conv_layer(fc4, fs4, nbv2, v2pad, f32(d4), 32, 1024, False)

    return _final(fm4, f32(Wl).T, f32(bl).reshape(1, -1), B, N2)


# speedup vs baseline: 43.3470x; 1.0066x over previous
import functools
import math

import numpy as _np

import jax
import jax.numpy as jnp
from jax import lax
from jax.experimental import pallas as pl
from jax.experimental.pallas import tpu as pltpu
from jax.experimental.pallas import tpu_sc as plsc

_SUP = 3
_NBR = 16
_PNBR = 4
_F32 = jnp.float32
_HI = lax.Precision.HIGHEST
_Z = _np.int32(0)


def _gather_rows(table, idx):
    T, D = table.shape
    (Btot,) = idx.shape
    info = plsc.get_sparse_core_info()
    NC, NS = info.num_cores, info.num_subcores
    NW = NC * NS
    assert Btot % (8 * NW) == 0 and D % 128 == 0
    rpw = Btot // NW
    cap = max(8, 180_000 // (4 * D))
    chunk = 8
    while chunk * 2 <= min(rpw, cap, 1024):
        chunk *= 2
    nchunks = rpw // chunk
    mesh = plsc.VectorSubcoreMesh(core_axis_name="c", subcore_axis_name="s")

    @functools.partial(
        pl.kernel,
        mesh=mesh,
        out_type=jax.ShapeDtypeStruct((Btot, D), _F32),
        scratch_types=[
            pltpu.VMEM((chunk,), jnp.int32),
            pltpu.VMEM((chunk,), jnp.int32),
            pltpu.VMEM((chunk, D), _F32),
            pltpu.VMEM((chunk, D), _F32),
            pltpu.SemaphoreType.DMA,
            pltpu.SemaphoreType.DMA,
            pltpu.SemaphoreType.DMA,
            pltpu.SemaphoreType.DMA,
        ],
    )
    def gk(table_hbm, idx_hbm, out_hbm, idx_a, idx_b, rows_a, rows_b,
           sga, sgb, swa, swb):
        i32 = jnp.int32
        wid = lax.axis_index("s") * i32(NC) + lax.axis_index("c")
        base0 = wid * i32(rpw)
        idx_v = (idx_a, idx_b)
        rows_v = (rows_a, rows_b)
        sg = (sga, sgb)
        sw = (swa, swb)

        def start_gather(c):
            b = c % 2
            base = base0 + i32(c * chunk)
            pltpu.sync_copy(idx_hbm.at[pl.ds(base, chunk)], idx_v[b])
            return pltpu.async_copy(table_hbm.at[idx_v[b]], rows_v[b], sg[b])

        gh = {0: start_gather(0)}
        if nchunks > 1:
            gh[1] = start_gather(1)
        wh = {}
        for c in range(nchunks):
            b = c % 2
            gh[c].wait()
            base = base0 + i32(c * chunk)
            wh[c] = pltpu.async_copy(rows_v[b], out_hbm.at[pl.ds(base, chunk)],
                                     sw[b])
            if c + 2 < nchunks:
                wh[c].wait()
                gh[c + 2] = start_gather(c + 2)
        for c in (nchunks - 2, nchunks - 1):
            if c >= 0 and c in wh and c + 2 >= nchunks:
                wh[c].wait()

    return gk(table, idx)


def _knn_global(vq, vtT, k, rq):
    B, Nq, _ = vq.shape
    Nt = vtT.shape[2]

    def body(vq_ref, vt_ref, o_ref):
        b = pl.program_id(0)
        q = vq_ref[0]
        t = vt_ref[0]
        inner = jnp.dot(q.astype(jnp.bfloat16), t.astype(jnp.bfloat16),
                        preferred_element_type=_F32)
        qq = q[:, 0:1] * q[:, 0:1]
        qt = t[0:1, :] * t[0:1, :]
        for d in (1, 2):
            qq = qq + q[:, d:d + 1] * q[:, d:d + 1]
            qt = qt + t[d:d + 1, :] * t[d:d + 1, :]
        dist = -2.0 * inner + qt + qq
        iota = lax.broadcasted_iota(jnp.int32, (rq, Nt), 1)
        big = jnp.int32(Nt)
        cols = []
        for _ in range(k):
            m = jnp.min(dist, axis=1, keepdims=True)
            am = jnp.min(jnp.where(dist <= m, iota, big), axis=1, keepdims=True)
            cols.append(am + b * Nt)
            dist = jnp.where(iota == am, _F32(jnp.inf), dist)
        o_ref[0] = jnp.concatenate(cols, axis=1)

    return pl.pallas_call(
        body,
        grid=(B, Nq // rq),
        in_specs=[
            pl.BlockSpec((1, rq, 3), lambda b, i: (b, i, _Z)),
            pl.BlockSpec((1, 3, Nt), lambda b, i: (b, _Z, _Z)),
        ],
        out_specs=pl.BlockSpec((1, rq, k), lambda b, i: (b, i, _Z)),
        out_shape=jax.ShapeDtypeStruct((B, Nq, k), jnp.int32),
    )(vq, vtT)


def _mm(x, w, b2d, rm):
    Rt, K = x.shape
    D = w.shape[1]

    def body(x_ref, w_ref, b_ref, o_ref):
        o_ref[...] = (
            jnp.dot(x_ref[...], w_ref[...], precision=_HI,
                    preferred_element_type=_F32)
            + b_ref[...]
        )

    return pl.pallas_call(
        body,
        grid=(Rt // rm,),
        in_specs=[
            pl.BlockSpec((rm, K), lambda i: (i, _Z)),
            pl.BlockSpec((K, D), lambda i: (_Z, _Z)),
            pl.BlockSpec((1, D), lambda i: (_Z, _Z)),
        ],
        out_specs=pl.BlockSpec((rm, D), lambda i: (i, _Z)),
        out_shape=jax.ShapeDtypeStruct((Rt, D), _F32),
    )(x, w, b2d)


def _dirs_norm(dirs):
    n2 = dirs[0:1, :] * dirs[0:1, :]
    for d in (1, 2):
        n2 = n2 + dirs[d:d + 1, :] * dirs[d:d + 1, :]
    return dirs / jnp.maximum(jnp.sqrt(n2), _F32(1e-12))


def _theta_j(nb_j, vq3, sdn):
    d = nb_j[:, 0:3] - vq3
    n2 = jnp.sum(d * d, axis=1, keepdims=True)
    dn = d / jnp.maximum(jnp.sqrt(n2), _F32(1e-12))
    th = (dn[:, 0:1] * sdn[0:1, :] + dn[:, 1:2] * sdn[1:2, :]
          + dn[:, 2:3] * sdn[2:3, :])
    return jnp.maximum(th, _F32(0.0))


def _conv_surface(nbv, vq, dirs, R, kn):
    Rt = vq.shape[0]
    D = dirs.shape[1]

    def body(nbv_ref, vq_ref, dir_ref, o_ref):
        sdn = _dirs_norm(dir_ref[...])
        vq3 = vq_ref[...][:, 0:3]
        m = _theta_j(nbv_ref[0], vq3, sdn)
        for j in range(1, _NBR):
            m = jnp.maximum(m, _theta_j(nbv_ref[j], vq3, sdn))
        acc = m[:, 0:kn]
        for s in range(1, _SUP):
            acc = acc + m[:, s * kn:(s + 1) * kn]
        o_ref[...] = jnp.maximum(acc, _F32(0.0))

    return pl.pallas_call(
        body,
        grid=(Rt // R,),
        in_specs=[
            pl.BlockSpec((_NBR, R, 128), lambda i: (_Z, i, _Z)),
            pl.BlockSpec((R, 128), lambda i: (i, _Z)),
            pl.BlockSpec((3, D), lambda i: (_Z, _Z)),
        ],
        out_specs=pl.BlockSpec((R, kn), lambda i: (i, _Z)),
        out_shape=jax.ShapeDtypeStruct((Rt, kn), _F32),
    )(nbv, vq, dirs)


def _conv_layer(fc, fs, nbv, vq, dirs, R, out, do_relu):
    Rt = vq.shape[0]
    D = dirs.shape[1]
    Dfull = fs.shape[2]

    def body(fc_ref, fs_ref, nbv_ref, vq_ref, dir_ref, o_ref):
        sdn = _dirs_norm(dir_ref[...])
        vq3 = vq_ref[...][:, 0:3]
        m = _theta_j(nbv_ref[0], vq3, sdn) * fs_ref[0][:, 0:D]
        for j in range(1, _NBR):
            m = jnp.maximum(
                m, _theta_j(nbv_ref[j], vq3, sdn) * fs_ref[j][:, 0:D])
        acc = fc_ref[...] + m[:, 0:out]
        for s in range(1, _SUP):
            acc = acc + m[:, s * out:(s + 1) * out]
        if do_relu:
            acc = jnp.maximum(acc, _F32(0.0))
        o_ref[...] = acc

    return pl.pallas_call(
        body,
        grid=(Rt // R,),
        in_specs=[
            pl.BlockSpec((R, out), lambda i: (i, _Z)),
            pl.BlockSpec((_NBR, R, Dfull), lambda i: (_Z, i, _Z)),
            pl.BlockSpec((_NBR, R, 128), lambda i: (_Z, i, _Z)),
            pl.BlockSpec((R, 128), lambda i: (i, _Z)),
            pl.BlockSpec((3, D), lambda i: (_Z, _Z)),
        ],
        out_specs=pl.BlockSpec((R, out), lambda i: (i, _Z)),
        out_shape=jax.ShapeDtypeStruct((Rt, out), _F32),
    )(fc, fs, nbv, vq, dirs)


def _maxpool4(rows, R):
    _, Rt, D = rows.shape

    def body(x_ref, o_ref):
        m = x_ref[0]
        for j in range(1, _PNBR):
            m = jnp.maximum(m, x_ref[j])
        o_ref[...] = m

    return pl.pallas_call(
        body,
        grid=(Rt // R,),
        in_specs=[pl.BlockSpec((_PNBR, R, D), lambda i: (_Z, i, _Z))],
        out_specs=pl.BlockSpec((R, D), lambda i: (i, _Z)),
        out_shape=jax.ShapeDtypeStruct((Rt, D), _F32),
    )(rows)


def _final(fm4, WlT, bl2d, B, N):
    D = fm4.shape[1]
    O = WlT.shape[1]

    def body(x_ref, w_ref, b_ref, o_ref):
        x3 = x_ref[...].reshape(B, N, D)
        fg = jnp.max(x3, axis=1)
        o_ref[...] = (
            jnp.dot(fg, w_ref[...], precision=_HI, preferred_element_type=_F32)
            + b_ref[...]
        )

    return pl.pallas_call(
        body,
        in_specs=[
            pl.BlockSpec((B * N, D), lambda: (_Z, _Z)),
            pl.BlockSpec((D, O), lambda: (_Z, _Z)),
            pl.BlockSpec((1, O), lambda: (_Z, _Z)),
        ],
        out_specs=pl.BlockSpec((B, O), lambda: (_Z, _Z)),
        out_shape=jax.ShapeDtypeStruct((B, O), _F32),
    )(fm4, WlT, bl2d)


def _padw(flat, w):
    R, d = flat.shape
    return jnp.concatenate([flat, jnp.zeros((R, w - d), _F32)], axis=1)


def kernel(vertices, dir0, w1, b1, d1, w2, b2, d2, w3, b3, d3, w4, b4, d4,
           Wl, bl):
    B, N0, _ = vertices.shape
    N1, N2 = N0 // 4, N0 // 16
    f32 = lambda x: x.astype(_F32)
    vertices = f32(vertices)

    sel1_g = jax.random.permutation(jax.random.key(1), N0)[:N1].astype(
        jnp.int32)
    sel2_g = jax.random.permutation(jax.random.key(2), N1)[:N2].astype(
        jnp.int32)

    w1r = f32(jnp.concatenate([w1[:, 64:], w1[:, :64]], axis=1))
    b1r = f32(jnp.concatenate([b1[64:], b1[:64]])).reshape(1, -1)

    def split(w, b, out):
        return (f32(w[:, :out]), f32(b[:out]).reshape(1, -1),
                f32(w[:, out:]), f32(b[out:]).reshape(1, -1))

    w2c, b2c, w2t, b2t = split(w2, b2, 128)
    w3c, b3c, w3t, b3t = split(w3, b3, 256)
    w4c, b4c, w4t, b4t = split(w4, b4, 1024)

    jmaj = lambda nbr: jnp.transpose(nbr, (2, 0, 1)).reshape(-1)

    dir0f, d1f, d2f, d3f, d4f = f32(dir0), f32(d1), f32(d2), f32(d3), f32(d4)

    def one_batch(v_b):
        vpad0 = _padw(v_b.reshape(N0, 3), 128)
        vtT0 = jnp.transpose(v_b, (0, 2, 1))

        nbr0 = _knn_global(v_b, vtT0, _NBR + 1, 256)[:, :, 1:]
        idx0 = jmaj(nbr0)
        nbv0 = _gather_rows(vpad0, idx0).reshape(_NBR, N0, 128)

        fm0 = _conv_surface(nbv0, vpad0, dir0f, 512, 32)
        fo1 = _mm(fm0, w1r, b1r, 1024)
        fc1 = fo1[:, 192:]
        fs1 = _gather_rows(fo1, idx0).reshape(_NBR, N0, 256)
        fm1 = _conv_layer(fc1, fs1, nbv0, vpad0, d1f, 256, 64, True)

        v1pad = _gather_rows(vpad0, sel1_g)
        v1 = v1pad[:, :3].reshape(1, N1, 3)
        nbrp1 = _knn_global(v1, vtT0, _PNBR + 1, 256)[:, :, 1:]
        prow1 = _gather_rows(_padw(fm1, 128), jmaj(nbrp1))
        fm1p = _maxpool4(prow1.reshape(_PNBR, N1, 128), 512)[:, :64]

        vtT1 = jnp.transpose(v1, (0, 2, 1))
        nbr1 = _knn_global(v1, vtT1, _NBR + 1, 256)[:, :, 1:]
        idx1 = jmaj(nbr1)
        nbv1 = _gather_rows(v1pad, idx1).reshape(_NBR, N1, 128)

        fc2 = _mm(fm1p, w2c, b2c, 1024)
        ft2 = _mm(fm1p, w2t, b2t, 1024)
        fs2 = _gather_rows(ft2, idx1).reshape(_NBR, N1, 384)
        fm2 = _conv_layer(fc2, fs2, nbv1, v1pad, d2f, 256, 128, True)

        fc3 = _mm(fm2, w3c, b3c, 1024)
        ft3 = _mm(fm2, w3t, b3t, 1024)
        fs3 = _gather_rows(ft3, idx1).reshape(_NBR, N1, 768)
        fm3 = _conv_layer(fc3, fs3, nbv1, v1pad, d3f, 128, 256, True)

        v2pad = _gather_rows(v1pad, sel2_g)
        v2 = v2pad[:, :3].reshape(1, N2, 3)
        nbrp2 = _knn_global(v2, vtT1, _PNBR + 1, 256)[:, :, 1:]
        prow2 = _gather_rows(fm3, jmaj(nbrp2))
        fm3p = _maxpool4(prow2.reshape(_PNBR, N2, 256), 256)

        vtT2 = jnp.transpose(v2, (0, 2, 1))
        nbr2 = _knn_global(v2, vtT2, _NBR + 1, 256)[:, :, 1:]
        idx2 = jmaj(nbr2)
        nbv2 = _gather_rows(v2pad, idx2).reshape(_NBR, N2, 128)

        fc4 = _mm(fm3p, w4c, b4c, 256)
        ft4 = _mm(fm3p, w4t, b4t, 256)
        fs4 = _gather_rows(ft4, idx2).reshape(_NBR, N2, 3072)
        return _conv_layer(fc4, fs4, nbv2, v2pad, d4f, 32, 1024, False)

    fm4 = jnp.concatenate([one_batch(vertices[b:b + 1]) for b in range(B)])
    return _final(fm4, f32(Wl).T, f32(bl).reshape(1, -1), B, N2)


# speedup vs baseline: 47.0124x; 1.0846x over previous
import functools
import math

import numpy as _np

import jax
import jax.numpy as jnp
from jax import lax
from jax.experimental import pallas as pl
from jax.experimental.pallas import tpu as pltpu
from jax.experimental.pallas import tpu_sc as plsc

_SUP = 3
_NBR = 16
_PNBR = 4
_F32 = jnp.float32
_HI = lax.Precision.HIGHEST
_Z = _np.int32(0)


def _gather_rows(table, idx):
    T, D = table.shape
    (Btot,) = idx.shape
    info = plsc.get_sparse_core_info()
    NC, NS = info.num_cores, info.num_subcores
    NW = NC * NS
    assert Btot % (8 * NW) == 0 and D % 128 == 0
    rpw = Btot // NW
    cap = max(8, 180_000 // (4 * D))
    chunk = 8
    while chunk * 2 <= min(rpw, cap, 1024):
        chunk *= 2
    nchunks = rpw // chunk
    mesh = plsc.VectorSubcoreMesh(core_axis_name="c", subcore_axis_name="s")

    @functools.partial(
        pl.kernel,
        mesh=mesh,
        out_type=jax.ShapeDtypeStruct((Btot, D), _F32),
        scratch_types=[
            pltpu.VMEM((chunk,), jnp.int32),
            pltpu.VMEM((chunk,), jnp.int32),
            pltpu.VMEM((chunk, D), _F32),
            pltpu.VMEM((chunk, D), _F32),
            pltpu.SemaphoreType.DMA,
            pltpu.SemaphoreType.DMA,
            pltpu.SemaphoreType.DMA,
            pltpu.SemaphoreType.DMA,
        ],
    )
    def gk(table_hbm, idx_hbm, out_hbm, idx_a, idx_b, rows_a, rows_b,
           sga, sgb, swa, swb):
        i32 = jnp.int32
        wid = lax.axis_index("s") * i32(NC) + lax.axis_index("c")
        base0 = wid * i32(rpw)
        idx_v = (idx_a, idx_b)
        rows_v = (rows_a, rows_b)
        sg = (sga, sgb)
        sw = (swa, swb)

        def start_gather(c):
            b = c % 2
            base = base0 + i32(c * chunk)
            pltpu.sync_copy(idx_hbm.at[pl.ds(base, chunk)], idx_v[b])
            return pltpu.async_copy(table_hbm.at[idx_v[b]], rows_v[b], sg[b])

        gh = {0: start_gather(0)}
        if nchunks > 1:
            gh[1] = start_gather(1)
        wh = {}
        for c in range(nchunks):
            b = c % 2
            gh[c].wait()
            base = base0 + i32(c * chunk)
            wh[c] = pltpu.async_copy(rows_v[b], out_hbm.at[pl.ds(base, chunk)],
                                     sw[b])
            if c + 2 < nchunks:
                wh[c].wait()
                gh[c + 2] = start_gather(c + 2)
        for c in (nchunks - 2, nchunks - 1):
            if c >= 0 and c in wh and c + 2 >= nchunks:
                wh[c].wait()

    return gk(table, idx)


def _knn_global(vq, vtT, k, rq):
    B, Nq, _ = vq.shape
    Nt = vtT.shape[2]

    def body(vq_ref, vt_ref, o_ref):
        b = pl.program_id(0)
        q = vq_ref[0]
        t = vt_ref[0]
        inner = jnp.dot(q.astype(jnp.bfloat16), t.astype(jnp.bfloat16),
                        preferred_element_type=_F32)
        qq = q[:, 0:1] * q[:, 0:1]
        qt = t[0:1, :] * t[0:1, :]
        for d in (1, 2):
            qq = qq + q[:, d:d + 1] * q[:, d:d + 1]
            qt = qt + t[d:d + 1, :] * t[d:d + 1, :]
        dist = -2.0 * inner + qt + qq
        iota = lax.broadcasted_iota(jnp.int32, (rq, Nt), 1)
        cols = []
        for it in range(k):
            am = lax.argmin(dist, 1, jnp.int32)[:, None]
            cols.append(am + b * Nt)
            if it + 1 < k:
                dist = jnp.where(iota == am, _F32(jnp.inf), dist)
        o_ref[0] = jnp.concatenate(cols, axis=1)

    return pl.pallas_call(
        body,
        grid=(B, Nq // rq),
        in_specs=[
            pl.BlockSpec((1, rq, 3), lambda b, i: (b, i, _Z)),
            pl.BlockSpec((1, 3, Nt), lambda b, i: (b, _Z, _Z)),
        ],
        out_specs=pl.BlockSpec((1, rq, k), lambda b, i: (b, i, _Z)),
        out_shape=jax.ShapeDtypeStruct((B, Nq, k), jnp.int32),
    )(vq, vtT)


def _mm(x, w, b2d, rm):
    Rt, K = x.shape
    D = w.shape[1]

    def body(x_ref, w_ref, b_ref, o_ref):
        o_ref[...] = (
            jnp.dot(x_ref[...], w_ref[...], precision=_HI,
                    preferred_element_type=_F32)
            + b_ref[...]
        )

    return pl.pallas_call(
        body,
        grid=(Rt // rm,),
        in_specs=[
            pl.BlockSpec((rm, K), lambda i: (i, _Z)),
            pl.BlockSpec((K, D), lambda i: (_Z, _Z)),
            pl.BlockSpec((1, D), lambda i: (_Z, _Z)),
        ],
        out_specs=pl.BlockSpec((rm, D), lambda i: (i, _Z)),
        out_shape=jax.ShapeDtypeStruct((Rt, D), _F32),
    )(x, w, b2d)


def _dirs_norm(dirs):
    n2 = dirs[0:1, :] * dirs[0:1, :]
    for d in (1, 2):
        n2 = n2 + dirs[d:d + 1, :] * dirs[d:d + 1, :]
    return dirs / jnp.maximum(jnp.sqrt(n2), _F32(1e-12))


def _theta_j(nb_j, vq3, sdn):
    d = nb_j[:, 0:3] - vq3
    n2 = jnp.sum(d * d, axis=1, keepdims=True)
    dn = d / jnp.maximum(jnp.sqrt(n2), _F32(1e-12))
    th = (dn[:, 0:1] * sdn[0:1, :] + dn[:, 1:2] * sdn[1:2, :]
          + dn[:, 2:3] * sdn[2:3, :])
    return jnp.maximum(th, _F32(0.0))


def _conv_surface(nbv, vq, dirs, R, kn):
    Rt = vq.shape[0]
    D = dirs.shape[1]

    def body(nbv_ref, vq_ref, dir_ref, o_ref):
        sdn = _dirs_norm(dir_ref[...])
        vq3 = vq_ref[...][:, 0:3]
        m = _theta_j(nbv_ref[0], vq3, sdn)
        for j in range(1, _NBR):
            m = jnp.maximum(m, _theta_j(nbv_ref[j], vq3, sdn))
        acc = m[:, 0:kn]
        for s in range(1, _SUP):
            acc = acc + m[:, s * kn:(s + 1) * kn]
        o_ref[...] = jnp.maximum(acc, _F32(0.0))

    return pl.pallas_call(
        body,
        grid=(Rt // R,),
        in_specs=[
            pl.BlockSpec((_NBR, R, 128), lambda i: (_Z, i, _Z)),
            pl.BlockSpec((R, 128), lambda i: (i, _Z)),
            pl.BlockSpec((3, D), lambda i: (_Z, _Z)),
        ],
        out_specs=pl.BlockSpec((R, kn), lambda i: (i, _Z)),
        out_shape=jax.ShapeDtypeStruct((Rt, kn), _F32),
    )(nbv, vq, dirs)


def _conv_layer(fc, fs, nbv, vq, dirs, R, out, do_relu):
    Rt = vq.shape[0]
    D = dirs.shape[1]
    Dfull = fs.shape[2]

    def body(fc_ref, fs_ref, nbv_ref, vq_ref, dir_ref, o_ref):
        sdn = _dirs_norm(dir_ref[...])
        vq3 = vq_ref[...][:, 0:3]
        m = _theta_j(nbv_ref[0], vq3, sdn) * fs_ref[0][:, 0:D]
        for j in range(1, _NBR):
            m = jnp.maximum(
                m, _theta_j(nbv_ref[j], vq3, sdn) * fs_ref[j][:, 0:D])
        acc = fc_ref[...] + m[:, 0:out]
        for s in range(1, _SUP):
            acc = acc + m[:, s * out:(s + 1) * out]
        if do_relu:
            acc = jnp.maximum(acc, _F32(0.0))
        o_ref[...] = acc

    return pl.pallas_call(
        body,
        grid=(Rt // R,),
        in_specs=[
            pl.BlockSpec((R, out), lambda i: (i, _Z)),
            pl.BlockSpec((_NBR, R, Dfull), lambda i: (_Z, i, _Z)),
            pl.BlockSpec((_NBR, R, 128), lambda i: (_Z, i, _Z)),
            pl.BlockSpec((R, 128), lambda i: (i, _Z)),
            pl.BlockSpec((3, D), lambda i: (_Z, _Z)),
        ],
        out_specs=pl.BlockSpec((R, out), lambda i: (i, _Z)),
        out_shape=jax.ShapeDtypeStruct((Rt, out), _F32),
    )(fc, fs, nbv, vq, dirs)


def _maxpool4(rows, R):
    _, Rt, D = rows.shape

    def body(x_ref, o_ref):
        m = x_ref[0]
        for j in range(1, _PNBR):
            m = jnp.maximum(m, x_ref[j])
        o_ref[...] = m

    return pl.pallas_call(
        body,
        grid=(Rt // R,),
        in_specs=[pl.BlockSpec((_PNBR, R, D), lambda i: (_Z, i, _Z))],
        out_specs=pl.BlockSpec((R, D), lambda i: (i, _Z)),
        out_shape=jax.ShapeDtypeStruct((Rt, D), _F32),
    )(rows)


def _final(fm4, WlT, bl2d, B, N):
    D = fm4.shape[1]
    O = WlT.shape[1]

    def body(x_ref, w_ref, b_ref, o_ref):
        x3 = x_ref[...].reshape(B, N, D)
        fg = jnp.max(x3, axis=1)
        o_ref[...] = (
            jnp.dot(fg, w_ref[...], precision=_HI, preferred_element_type=_F32)
            + b_ref[...]
        )

    return pl.pallas_call(
        body,
        in_specs=[
            pl.BlockSpec((B * N, D), lambda: (_Z, _Z)),
            pl.BlockSpec((D, O), lambda: (_Z, _Z)),
            pl.BlockSpec((1, O), lambda: (_Z, _Z)),
        ],
        out_specs=pl.BlockSpec((B, O), lambda: (_Z, _Z)),
        out_shape=jax.ShapeDtypeStruct((B, O), _F32),
    )(fm4, WlT, bl2d)


def _padw(flat, w):
    R, d = flat.shape
    return jnp.concatenate([flat, jnp.zeros((R, w - d), _F32)], axis=1)


def kernel(vertices, dir0, w1, b1, d1, w2, b2, d2, w3, b3, d3, w4, b4, d4,
           Wl, bl):
    B, N0, _ = vertices.shape
    N1, N2 = N0 // 4, N0 // 16
    f32 = lambda x: x.astype(_F32)
    vertices = f32(vertices)

    sel1_g = jax.random.permutation(jax.random.key(1), N0)[:N1].astype(
        jnp.int32)
    sel2_g = jax.random.permutation(jax.random.key(2), N1)[:N2].astype(
        jnp.int32)

    w1r = f32(jnp.concatenate([w1[:, 64:], w1[:, :64]], axis=1))
    b1r = f32(jnp.concatenate([b1[64:], b1[:64]])).reshape(1, -1)

    def split(w, b, out):
        return (f32(w[:, :out]), f32(b[:out]).reshape(1, -1),
                f32(w[:, out:]), f32(b[out:]).reshape(1, -1))

    w2c, b2c, w2t, b2t = split(w2, b2, 128)
    w3c, b3c, w3t, b3t = split(w3, b3, 256)
    w4c, b4c, w4t, b4t = split(w4, b4, 1024)

    jmaj = lambda nbr: jnp.transpose(nbr, (2, 0, 1)).reshape(-1)

    dir0f, d1f, d2f, d3f, d4f = f32(dir0), f32(d1), f32(d2), f32(d3), f32(d4)

    def one_batch(v_b):
        vpad0 = _padw(v_b.reshape(N0, 3), 128)
        vtT0 = jnp.transpose(v_b, (0, 2, 1))

        nbr0 = _knn_global(v_b, vtT0, _NBR + 1, 256)[:, :, 1:]
        idx0 = jmaj(nbr0)
        nbv0 = _gather_rows(vpad0, idx0).reshape(_NBR, N0, 128)

        fm0 = _conv_surface(nbv0, vpad0, dir0f, 512, 32)
        fo1 = _mm(fm0, w1r, b1r, 1024)
        fc1 = fo1[:, 192:]
        fs1 = _gather_rows(fo1, idx0).reshape(_NBR, N0, 256)
        fm1 = _conv_layer(fc1, fs1, nbv0, vpad0, d1f, 256, 64, True)

        v1pad = _gather_rows(vpad0, sel1_g)
        v1 = v1pad[:, :3].reshape(1, N1, 3)
        nbrp1 = _knn_global(v1, vtT0, _PNBR + 1, 256)[:, :, 1:]
        prow1 = _gather_rows(_padw(fm1, 128), jmaj(nbrp1))
        fm1p = _maxpool4(prow1.reshape(_PNBR, N1, 128), 512)[:, :64]

        vtT1 = jnp.transpose(v1, (0, 2, 1))
        nbr1 = _knn_global(v1, vtT1, _NBR + 1, 256)[:, :, 1:]
        idx1 = jmaj(nbr1)
        nbv1 = _gather_rows(v1pad, idx1).reshape(_NBR, N1, 128)

        fc2 = _mm(fm1p, w2c, b2c, 1024)
        ft2 = _mm(fm1p, w2t, b2t, 1024)
        fs2 = _gather_rows(ft2, idx1).reshape(_NBR, N1, 384)
        fm2 = _conv_layer(fc2, fs2, nbv1, v1pad, d2f, 256, 128, True)

        fc3 = _mm(fm2, w3c, b3c, 1024)
        ft3 = _mm(fm2, w3t, b3t, 1024)
        fs3 = _gather_rows(ft3, idx1).reshape(_NBR, N1, 768)
        fm3 = _conv_layer(fc3, fs3, nbv1, v1pad, d3f, 128, 256, True)

        v2pad = _gather_rows(v1pad, sel2_g)
        v2 = v2pad[:, :3].reshape(1, N2, 3)
        nbrp2 = _knn_global(v2, vtT1, _PNBR + 1, 256)[:, :, 1:]
        prow2 = _gather_rows(fm3, jmaj(nbrp2))
        fm3p = _maxpool4(prow2.reshape(_PNBR, N2, 256), 256)

        vtT2 = jnp.transpose(v2, (0, 2, 1))
        nbr2 = _knn_global(v2, vtT2, _NBR + 1, 256)[:, :, 1:]
        idx2 = jmaj(nbr2)
        nbv2 = _gather_rows(v2pad, idx2).reshape(_NBR, N2, 128)

        fc4 = _mm(fm3p, w4c, b4c, 256)
        ft4 = _mm(fm3p, w4t, b4t, 256)
        fs4 = _gather_rows(ft4, idx2).reshape(_NBR, N2, 3072)
        return _conv_layer(fc4, fs4, nbv2, v2pad, d4f, 32, 1024, False)

    fm4 = jnp.concatenate([one_batch(vertices[b:b + 1]) for b in range(B)])
    return _final(fm4, f32(Wl).T, f32(bl).reshape(1, -1), B, N2)


# speedup vs baseline: 47.6877x; 1.0144x over previous
import functools
import math

import numpy as _np

import jax
import jax.numpy as jnp
from jax import lax
from jax.experimental import pallas as pl
from jax.experimental.pallas import tpu as pltpu
from jax.experimental.pallas import tpu_sc as plsc

_SUP = 3
_NBR = 16
_PNBR = 4
_F32 = jnp.float32
_HI = lax.Precision.HIGHEST
_Z = _np.int32(0)


def _gather_rows(table, idx):
    T, D = table.shape
    (Btot,) = idx.shape
    info = plsc.get_sparse_core_info()
    NC, NS = info.num_cores, info.num_subcores
    NW = NC * NS
    assert Btot % (8 * NW) == 0 and D % 128 == 0
    rpw = Btot // NW
    NBUF = 3
    cap = max(8, 440_000 // (4 * D * NBUF))
    chunk = 8
    while chunk * 2 <= min(rpw, cap, 1024):
        chunk *= 2
    nchunks = rpw // chunk
    nbuf = min(NBUF, nchunks)
    mesh = plsc.VectorSubcoreMesh(core_axis_name="c", subcore_axis_name="s")

    @functools.partial(
        pl.kernel,
        mesh=mesh,
        out_type=jax.ShapeDtypeStruct((Btot, D), _F32),
        scratch_types=(
            [pltpu.VMEM((rpw,), jnp.int32)]
            + [pltpu.VMEM((chunk, D), _F32)] * NBUF
            + [pltpu.SemaphoreType.DMA] * (2 * NBUF)
        ),
    )
    def gk(table_hbm, idx_hbm, out_hbm, idx_all, r0, r1, r2, *sems):
        i32 = jnp.int32
        wid = lax.axis_index("s") * i32(NC) + lax.axis_index("c")
        base0 = wid * i32(rpw)
        rows_v = (r0, r1, r2)
        sg, sw = sems[:NBUF], sems[NBUF:]
        pltpu.sync_copy(idx_hbm.at[pl.ds(base0, rpw)], idx_all)

        def start_gather(c):
            b = c % nbuf
            return pltpu.async_copy(
                table_hbm.at[idx_all.at[pl.ds(c * chunk, chunk)]],
                rows_v[b], sg[b])

        gh = {c: start_gather(c) for c in range(min(nbuf, nchunks))}
        wh = {}
        for c in range(nchunks):
            b = c % nbuf
            gh[c].wait()
            base = base0 + i32(c * chunk)
            wh[c] = pltpu.async_copy(rows_v[b], out_hbm.at[pl.ds(base, chunk)],
                                     sw[b])
            if c + nbuf < nchunks:
                wh[c].wait()
                gh[c + nbuf] = start_gather(c + nbuf)
        for c in wh:
            if c + nbuf >= nchunks:
                wh[c].wait()

    return gk(table, idx)


def _knn_global(vq, vtT, k, rq):
    B, Nq, _ = vq.shape
    Nt = vtT.shape[2]

    def body(vq_ref, vt_ref, o_ref):
        b = pl.program_id(0)
        q = vq_ref[0]
        t = vt_ref[0]
        inner = jnp.dot(q.astype(jnp.bfloat16), t.astype(jnp.bfloat16),
                        preferred_element_type=_F32)
        qq = q[:, 0:1] * q[:, 0:1]
        qt = t[0:1, :] * t[0:1, :]
        for d in (1, 2):
            qq = qq + q[:, d:d + 1] * q[:, d:d + 1]
            qt = qt + t[d:d + 1, :] * t[d:d + 1, :]
        dist = -2.0 * inner + qt + qq
        iota = lax.broadcasted_iota(jnp.int32, (rq, Nt), 1)
        cols = []
        for it in range(k):
            am = lax.argmin(dist, 1, jnp.int32)[:, None]
            cols.append(am + b * Nt)
            if it + 1 < k:
                dist = jnp.where(iota == am, _F32(jnp.inf), dist)
        o_ref[0] = jnp.concatenate(cols, axis=1)

    return pl.pallas_call(
        body,
        grid=(B, Nq // rq),
        in_specs=[
            pl.BlockSpec((1, rq, 3), lambda b, i: (b, i, _Z)),
            pl.BlockSpec((1, 3, Nt), lambda b, i: (b, _Z, _Z)),
        ],
        out_specs=pl.BlockSpec((1, rq, k), lambda b, i: (b, i, _Z)),
        out_shape=jax.ShapeDtypeStruct((B, Nq, k), jnp.int32),
    )(vq, vtT)


def _mm(x, w, b2d, rm):
    Rt, K = x.shape
    D = w.shape[1]

    def body(x_ref, w_ref, b_ref, o_ref):
        o_ref[...] = (
            jnp.dot(x_ref[...], w_ref[...], precision=_HI,
                    preferred_element_type=_F32)
            + b_ref[...]
        )

    return pl.pallas_call(
        body,
        grid=(Rt // rm,),
        in_specs=[
            pl.BlockSpec((rm, K), lambda i: (i, _Z)),
            pl.BlockSpec((K, D), lambda i: (_Z, _Z)),
            pl.BlockSpec((1, D), lambda i: (_Z, _Z)),
        ],
        out_specs=pl.BlockSpec((rm, D), lambda i: (i, _Z)),
        out_shape=jax.ShapeDtypeStruct((Rt, D), _F32),
    )(x, w, b2d)


def _dirs_norm(dirs):
    n2 = dirs[0:1, :] * dirs[0:1, :]
    for d in (1, 2):
        n2 = n2 + dirs[d:d + 1, :] * dirs[d:d + 1, :]
    return dirs / jnp.maximum(jnp.sqrt(n2), _F32(1e-12))


def _theta_j(nb_j, vq3, sdn):
    d = nb_j[:, 0:3] - vq3
    n2 = jnp.sum(d * d, axis=1, keepdims=True)
    dn = d / jnp.maximum(jnp.sqrt(n2), _F32(1e-12))
    th = (dn[:, 0:1] * sdn[0:1, :] + dn[:, 1:2] * sdn[1:2, :]
          + dn[:, 2:3] * sdn[2:3, :])
    return jnp.maximum(th, _F32(0.0))


def _conv_surface(nbv, vq, dirs, R, kn):
    Rt = vq.shape[0]
    D = dirs.shape[1]

    def body(nbv_ref, vq_ref, dir_ref, o_ref):
        sdn = _dirs_norm(dir_ref[...])
        vq3 = vq_ref[...][:, 0:3]
        m = _theta_j(nbv_ref[0], vq3, sdn)
        for j in range(1, _NBR):
            m = jnp.maximum(m, _theta_j(nbv_ref[j], vq3, sdn))
        acc = m[:, 0:kn]
        for s in range(1, _SUP):
            acc = acc + m[:, s * kn:(s + 1) * kn]
        o_ref[...] = jnp.maximum(acc, _F32(0.0))

    return pl.pallas_call(
        body,
        grid=(Rt // R,),
        in_specs=[
            pl.BlockSpec((_NBR, R, 128), lambda i: (_Z, i, _Z)),
            pl.BlockSpec((R, 128), lambda i: (i, _Z)),
            pl.BlockSpec((3, D), lambda i: (_Z, _Z)),
        ],
        out_specs=pl.BlockSpec((R, kn), lambda i: (i, _Z)),
        out_shape=jax.ShapeDtypeStruct((Rt, kn), _F32),
    )(nbv, vq, dirs)


def _conv_layer(fc, fs, nbv, vq, dirs, R, out, do_relu):
    Rt = vq.shape[0]
    D = dirs.shape[1]
    Dfull = fs.shape[2]

    def body(fc_ref, fs_ref, nbv_ref, vq_ref, dir_ref, o_ref):
        sdn = _dirs_norm(dir_ref[...])
        vq3 = vq_ref[...][:, 0:3]
        m = _theta_j(nbv_ref[0], vq3, sdn) * fs_ref[0][:, 0:D]
        for j in range(1, _NBR):
            m = jnp.maximum(
                m, _theta_j(nbv_ref[j], vq3, sdn) * fs_ref[j][:, 0:D])
        acc = fc_ref[...] + m[:, 0:out]
        for s in range(1, _SUP):
            acc = acc + m[:, s * out:(s + 1) * out]
        if do_relu:
            acc = jnp.maximum(acc, _F32(0.0))
        o_ref[...] = acc

    return pl.pallas_call(
        body,
        grid=(Rt // R,),
        in_specs=[
            pl.BlockSpec((R, out), lambda i: (i, _Z)),
            pl.BlockSpec((_NBR, R, Dfull), lambda i: (_Z, i, _Z)),
            pl.BlockSpec((_NBR, R, 128), lambda i: (_Z, i, _Z)),
            pl.BlockSpec((R, 128), lambda i: (i, _Z)),
            pl.BlockSpec((3, D), lambda i: (_Z, _Z)),
        ],
        out_specs=pl.BlockSpec((R, out), lambda i: (i, _Z)),
        out_shape=jax.ShapeDtypeStruct((Rt, out), _F32),
    )(fc, fs, nbv, vq, dirs)


def _maxpool4(rows, R):
    _, Rt, D = rows.shape

    def body(x_ref, o_ref):
        m = x_ref[0]
        for j in range(1, _PNBR):
            m = jnp.maximum(m, x_ref[j])
        o_ref[...] = m

    return pl.pallas_call(
        body,
        grid=(Rt // R,),
        in_specs=[pl.BlockSpec((_PNBR, R, D), lambda i: (_Z, i, _Z))],
        out_specs=pl.BlockSpec((R, D), lambda i: (i, _Z)),
        out_shape=jax.ShapeDtypeStruct((Rt, D), _F32),
    )(rows)


def _final(fm4, WlT, bl2d, B, N):
    D = fm4.shape[1]
    O = WlT.shape[1]

    def body(x_ref, w_ref, b_ref, o_ref):
        x3 = x_ref[...].reshape(B, N, D)
        fg = jnp.max(x3, axis=1)
        o_ref[...] = (
            jnp.dot(fg, w_ref[...], precision=_HI, preferred_element_type=_F32)
            + b_ref[...]
        )

    return pl.pallas_call(
        body,
        in_specs=[
            pl.BlockSpec((B * N, D), lambda: (_Z, _Z)),
            pl.BlockSpec((D, O), lambda: (_Z, _Z)),
            pl.BlockSpec((1, O), lambda: (_Z, _Z)),
        ],
        out_specs=pl.BlockSpec((B, O), lambda: (_Z, _Z)),
        out_shape=jax.ShapeDtypeStruct((B, O), _F32),
    )(fm4, WlT, bl2d)


def _padw(flat, w):
    R, d = flat.shape
    return jnp.concatenate([flat, jnp.zeros((R, w - d), _F32)], axis=1)


def kernel(vertices, dir0, w1, b1, d1, w2, b2, d2, w3, b3, d3, w4, b4, d4,
           Wl, bl):
    B, N0, _ = vertices.shape
    N1, N2 = N0 // 4, N0 // 16
    f32 = lambda x: x.astype(_F32)
    vertices = f32(vertices)

    sel1_g = jax.random.permutation(jax.random.key(1), N0)[:N1].astype(
        jnp.int32)
    sel2_g = jax.random.permutation(jax.random.key(2), N1)[:N2].astype(
        jnp.int32)

    w1r = f32(jnp.concatenate([w1[:, 64:], w1[:, :64]], axis=1))
    b1r = f32(jnp.concatenate([b1[64:], b1[:64]])).reshape(1, -1)

    def split(w, b, out):
        return (f32(w[:, :out]), f32(b[:out]).reshape(1, -1),
                f32(w[:, out:]), f32(b[out:]).reshape(1, -1))

    w2c, b2c, w2t, b2t = split(w2, b2, 128)
    w3c, b3c, w3t, b3t = split(w3, b3, 256)
    w4c, b4c, w4t, b4t = split(w4, b4, 1024)

    jmaj = lambda nbr: jnp.transpose(nbr, (2, 0, 1)).reshape(-1)

    dir0f, d1f, d2f, d3f, d4f = f32(dir0), f32(d1), f32(d2), f32(d3), f32(d4)

    def one_batch(v_b):
        vpad0 = _padw(v_b.reshape(N0, 3), 128)
        vtT0 = jnp.transpose(v_b, (0, 2, 1))

        nbr0 = _knn_global(v_b, vtT0, _NBR + 1, 256)[:, :, 1:]
        idx0 = jmaj(nbr0)
        nbv0 = _gather_rows(vpad0, idx0).reshape(_NBR, N0, 128)

        fm0 = _conv_surface(nbv0, vpad0, dir0f, 512, 32)
        fo1 = _mm(fm0, w1r, b1r, 1024)
        fc1 = fo1[:, 192:]
        fs1 = _gather_rows(fo1, idx0).reshape(_NBR, N0, 256)
        fm1 = _conv_layer(fc1, fs1, nbv0, vpad0, d1f, 256, 64, True)

        v1pad = _gather_rows(vpad0, sel1_g)
        v1 = v1pad[:, :3].reshape(1, N1, 3)
        nbrp1 = _knn_global(v1, vtT0, _PNBR + 1, 256)[:, :, 1:]
        prow1 = _gather_rows(_padw(fm1, 128), jmaj(nbrp1))
        fm1p = _maxpool4(prow1.reshape(_PNBR, N1, 128), 512)[:, :64]

        vtT1 = jnp.transpose(v1, (0, 2, 1))
        nbr1 = _knn_global(v1, vtT1, _NBR + 1, 256)[:, :, 1:]
        idx1 = jmaj(nbr1)
        nbv1 = _gather_rows(v1pad, idx1).reshape(_NBR, N1, 128)

        fc2 = _mm(fm1p, w2c, b2c, 1024)
        ft2 = _mm(fm1p, w2t, b2t, 1024)
        fs2 = _gather_rows(ft2, idx1).reshape(_NBR, N1, 384)
        fm2 = _conv_layer(fc2, fs2, nbv1, v1pad, d2f, 256, 128, True)

        fc3 = _mm(fm2, w3c, b3c, 1024)
        ft3 = _mm(fm2, w3t, b3t, 1024)
        fs3 = _gather_rows(ft3, idx1).reshape(_NBR, N1, 768)
        fm3 = _conv_layer(fc3, fs3, nbv1, v1pad, d3f, 128, 256, True)

        v2pad = _gather_rows(v1pad, sel2_g)
        v2 = v2pad[:, :3].reshape(1, N2, 3)
        nbrp2 = _knn_global(v2, vtT1, _PNBR + 1, 256)[:, :, 1:]
        prow2 = _gather_rows(fm3, jmaj(nbrp2))
        fm3p = _maxpool4(prow2.reshape(_PNBR, N2, 256), 256)

        vtT2 = jnp.transpose(v2, (0, 2, 1))
        nbr2 = _knn_global(v2, vtT2, _NBR + 1, 256)[:, :, 1:]
        idx2 = jmaj(nbr2)
        nbv2 = _gather_rows(v2pad, idx2).reshape(_NBR, N2, 128)

        fc4 = _mm(fm3p, w4c, b4c, 256)
        ft4 = _mm(fm3p, w4t, b4t, 256)
        fs4 = _gather_rows(ft4, idx2).reshape(_NBR, N2, 3072)
        return _conv_layer(fc4, fs4, nbv2, v2pad, d4f, 32, 1024, False)

    fm4 = jnp.concatenate([one_batch(vertices[b:b + 1]) for b in range(B)])
    return _final(fm4, f32(Wl).T, f32(bl).reshape(1, -1), B, N2)
